# baseline (device time: 7079 ns/iter reference)
import jax
import jax.numpy as jnp
from jax import lax
from jax.experimental import pallas as pl
from jax.experimental.pallas import tpu as pltpu

K = 8
NEG_INF = float("-inf")

_BITONIC8 = [(0, 4), (1, 5), (2, 6), (3, 7),
             (0, 2), (1, 3), (4, 6), (5, 7),
             (0, 1), (2, 3), (4, 5), (6, 7)]


def kernel(x):
    m, n = x.shape
    h = m // 2

    def body(x_ref, out_ref, cand_ref, send_sems, recv_sems):
        my_x = lax.axis_index("x")
        my_y = lax.axis_index("y")
        peer = (my_x, 1 - my_y)

        barrier_sem = pltpu.get_barrier_semaphore()
        pl.semaphore_signal(
            barrier_sem, inc=1, device_id=peer,
            device_id_type=pl.DeviceIdType.MESH,
        )

        def ce(u, v):
            return jnp.maximum(u, v), jnp.minimum(u, v)

        def local_topk(rows):
            q = n // 4
            a = x_ref[rows, 0 * q:1 * q]
            b = x_ref[rows, 1 * q:2 * q]
            c = x_ref[rows, 2 * q:3 * q]
            d = x_ref[rows, 3 * q:4 * q]
            p0, p1 = ce(a, b)
            p2, p3 = ce(c, d)
            s0, t2 = ce(p0, p2)
            t1, s3 = ce(p1, p3)
            s1, s2 = ce(t1, t2)
            cols = []
            for _ in range(K):
                mx = jnp.max(s0, axis=1, keepdims=True)
                cols.append(mx)
                hit = s0 == mx
                s0 = jnp.where(hit, s1, s0)
                s1 = jnp.where(hit, s2, s1)
                s2 = jnp.where(hit, s3, s2)
                s3 = jnp.where(hit, NEG_INF, s3)
            return jnp.concatenate(cols, axis=1).T

        def exchange(blk):
            rdma = pltpu.make_async_remote_copy(
                src_ref=cand_ref.at[0, blk],
                dst_ref=cand_ref.at[1, blk],
                send_sem=send_sems.at[blk],
                recv_sem=recv_sems.at[blk],
                device_id=peer,
                device_id_type=pl.DeviceIdType.MESH,
            )
            rdma.start()
            return rdma

        def merge_store(blk, rows):
            L = [
                jnp.maximum(cand_ref[0, blk, i, :],
                            cand_ref[1, blk, K - 1 - i, :])
                for i in range(K)
            ]
            for (i, j) in _BITONIC8:
                L[i], L[j] = jnp.maximum(L[i], L[j]), jnp.minimum(L[i], L[j])
            out_ref[rows, :] = jnp.stack(L, axis=0).T

        cand_ref[0, 0, :, :] = local_topk(pl.ds(0, h))
        pl.semaphore_wait(barrier_sem, 1)
        rdma_a = exchange(0)

        cand_ref[0, 1, :, :] = local_topk(pl.ds(h, h))
        rdma_b = exchange(1)

        rdma_a.wait()
        merge_store(0, pl.ds(0, h))
        rdma_b.wait()
        merge_store(1, pl.ds(h, h))

    return pl.pallas_call(
        body,
        out_shape=jax.ShapeDtypeStruct((m, K), jnp.float32),
        in_specs=[pl.BlockSpec(memory_space=pltpu.VMEM)],
        out_specs=pl.BlockSpec(memory_space=pltpu.VMEM),
        scratch_shapes=[
            pltpu.VMEM((2, 2, K, h), jnp.float32),
            pltpu.SemaphoreType.DMA((2,)),
            pltpu.SemaphoreType.DMA((2,)),
        ],
        compiler_params=pltpu.CompilerParams(collective_id=0),
    )(x)


# device time: 6518 ns/iter; 1.0861x vs baseline; 1.0861x over previous
import jax
import jax.numpy as jnp
from jax import lax
from jax.experimental import pallas as pl
from jax.experimental.pallas import tpu as pltpu

K = 8
NEG_INF = float("-inf")


def kernel(x):
    m, n = x.shape

    def body(x_ref, out_ref, cand_ref, send_sem, recv_sem):
        my_x = lax.axis_index("x")
        my_y = lax.axis_index("y")
        peer = (my_x, 1 - my_y)

        barrier_sem = pltpu.get_barrier_semaphore()
        pl.semaphore_signal(
            barrier_sem, inc=1, device_id=peer,
            device_id_type=pl.DeviceIdType.MESH,
        )

        q = n // 4
        a = x_ref[:, 0 * q:1 * q]
        b = x_ref[:, 1 * q:2 * q]
        c = x_ref[:, 2 * q:3 * q]
        d = x_ref[:, 3 * q:4 * q]

        def ce(u, v):
            return jnp.maximum(u, v), jnp.minimum(u, v)

        p0, p1 = ce(a, b)
        p2, p3 = ce(c, d)
        s0, t2 = ce(p0, p2)
        t1, s3 = ce(p1, p3)
        s1, s2 = ce(t1, t2)

        cols = []
        for _ in range(K):
            mx = jnp.max(s0, axis=1, keepdims=True)
            cols.append(mx)
            hit = s0 == mx
            s0 = jnp.where(hit, s1, s0)
            s1 = jnp.where(hit, s2, s1)
            s2 = jnp.where(hit, s3, s2)
            s3 = jnp.where(hit, NEG_INF, s3)
        local_top = jnp.concatenate(cols, axis=1)
        cand_ref[0, :, :] = local_top.T

        pl.semaphore_wait(barrier_sem, 1)

        rdma = pltpu.make_async_remote_copy(
            src_ref=cand_ref.at[0],
            dst_ref=cand_ref.at[1],
            send_sem=send_sem,
            recv_sem=recv_sem,
            device_id=peer,
            device_id_type=pl.DeviceIdType.MESH,
        )
        rdma.start()
        rdma.wait()

        L = [
            jnp.maximum(cand_ref[0, i, :], cand_ref[1, K - 1 - i, :])
            for i in range(K)
        ]

        def merge_ce(i, j):
            L[i], L[j] = jnp.maximum(L[i], L[j]), jnp.minimum(L[i], L[j])

        for (i, j) in [(0, 4), (1, 5), (2, 6), (3, 7),
                       (0, 2), (1, 3), (4, 6), (5, 7),
                       (0, 1), (2, 3), (4, 5), (6, 7)]:
            merge_ce(i, j)
        out_ref[:, :] = jnp.stack(L, axis=0).T

    return pl.pallas_call(
        body,
        out_shape=jax.ShapeDtypeStruct((m, K), jnp.float32),
        in_specs=[pl.BlockSpec(memory_space=pltpu.VMEM)],
        out_specs=pl.BlockSpec(memory_space=pltpu.VMEM),
        scratch_shapes=[
            pltpu.VMEM((2, K, m), jnp.float32),
            pltpu.SemaphoreType.DMA,
            pltpu.SemaphoreType.DMA,
        ],
        compiler_params=pltpu.CompilerParams(collective_id=0),
    )(x)
